# baseline (speedup 1.0000x reference)
import functools

import jax
import jax.numpy as jnp
from jax import lax
from jax.experimental import pallas as pl
from jax.experimental.pallas import tpu as pltpu
from jax.experimental.pallas import tpu_sc as plsc

_I = 3
_J = 700
_ROWS = 16384
_COLS = 1024
_NW = 32
_RPW = _ROWS // _NW
_CH = 32
_NCH = _RPW // _CH


def _cs_body(theta_ref, o_ref):
    t = theta_ref[0]
    c = jnp.cos(2.0 * t)
    s = jnp.sin(2.0 * t)
    o_ref[...] = jnp.concatenate(
        [jnp.full((4, 128), c, jnp.float32), jnp.full((4, 128), s, jnp.float32)])


def _sc_body(x_hbm, cs_hbm, o_hbm, b0, b1, csv, si0, si1, so0, so1):
    wid = lax.axis_index("s") * 2 + lax.axis_index("c")
    base = wid * _RPW
    pltpu.sync_copy(cs_hbm, csv)
    c16 = csv[0, 0:16]
    s16 = csv[4, 0:16]
    bufs = (b0, b1)
    isems = (si0, si1)
    osems = (so0, so1)

    def in_cp(k):
        return pltpu.make_async_copy(
            x_hbm.at[pl.ds(base + k * _CH, _CH), :], bufs[k % 2], isems[k % 2])

    def out_cp(k):
        return pltpu.make_async_copy(
            bufs[k % 2], o_hbm.at[pl.ds(base + k * _CH, _CH), :], osems[k % 2])

    def fix(buf):
        ci = jnp.full((16,), _I, jnp.int32)
        cj = jnp.full((16,), _J, jnp.int32)
        for g in range(_CH // 16):
            rows = jax.lax.iota(jnp.int32, 16) + (g * 16)
            xi = plsc.load_gather(buf, [rows, ci])
            xj = plsc.load_gather(buf, [rows, cj])
            plsc.store_scatter(buf, [rows, ci], xi * c16 + xj * s16)
            plsc.store_scatter(buf, [rows, cj], xi * s16 - xj * c16)

    in_cp(0).start()
    for k in range(_NCH):
        in_cp(k).wait()
        fix(bufs[k % 2])
        out_cp(k).start()
        if k + 1 < _NCH:
            if k >= 1:
                out_cp(k - 1).wait()
            in_cp(k + 1).start()
    out_cp(_NCH - 2).wait()
    out_cp(_NCH - 1).wait()


def kernel(x, theta):
    theta_arr = jnp.reshape(theta, (1,)).astype(jnp.float32)
    cs = pl.pallas_call(
        _cs_body,
        in_specs=[pl.BlockSpec(memory_space=pltpu.SMEM)],
        out_shape=jax.ShapeDtypeStruct((8, 128), jnp.float32),
    )(theta_arr)
    mesh = plsc.VectorSubcoreMesh(core_axis_name="c", subcore_axis_name="s")
    sc = functools.partial(
        pl.kernel,
        out_type=jax.ShapeDtypeStruct((_ROWS, _COLS), jnp.float32),
        mesh=mesh,
        compiler_params=pltpu.CompilerParams(needs_layout_passes=False),
        scratch_types=[
            pltpu.VMEM((_CH, _COLS), jnp.float32),
            pltpu.VMEM((_CH, _COLS), jnp.float32),
            pltpu.VMEM((8, 128), jnp.float32),
            pltpu.SemaphoreType.DMA,
            pltpu.SemaphoreType.DMA,
            pltpu.SemaphoreType.DMA,
            pltpu.SemaphoreType.DMA,
        ],
    )(_sc_body)
    return sc(x, cs)


# speedup vs baseline: 1.6769x; 1.6769x over previous
import jax
import jax.numpy as jnp
from jax.experimental import pallas as pl
from jax.experimental.pallas import tpu as pltpu

_I = 3
_J = 700
_ROWS = 16384
_COLS = 1024
_CHUNKS = (1024, 4096, 4096, 4096, 2048, 1024)
_OFFS = tuple(sum(_CHUNKS[:i]) for i in range(len(_CHUNKS)))
_N = len(_CHUNKS)
_BUF = (3, 0, 1, 2, 0, 3)
_BUFSHAPES = (4096, 4096, 4096, 1024)
_PREV = tuple(
    max((j for j in range(i) if _BUF[j] == _BUF[i]), default=None)
    for i in range(_N))


def _body(theta_ref, x_ref, o_ref, b0, b1, b2, b3, in_sems, out_sems):
    bufs = (b0, b1, b2, b3)
    t = theta_ref[0]
    c = jnp.cos(2.0 * t)
    s = jnp.sin(2.0 * t)

    def in_cp(i):
        return pltpu.make_async_copy(
            x_ref.at[pl.ds(_OFFS[i], _CHUNKS[i]), :],
            bufs[_BUF[i]].at[pl.ds(0, _CHUNKS[i]), :], in_sems.at[i])

    def out_cp(i):
        return pltpu.make_async_copy(
            bufs[_BUF[i]].at[pl.ds(0, _CHUNKS[i]), :],
            o_ref.at[pl.ds(_OFFS[i], _CHUNKS[i]), :], out_sems.at[i])

    for i in range(_N):
        if _PREV[i] is None:
            in_cp(i).start()
    for i in range(_N):
        buf = bufs[_BUF[i]]
        r = _CHUNKS[i]
        in_cp(i).wait()
        xi = buf[0:r, _I:_I + 1]
        xj = buf[0:r, _J:_J + 1]
        buf[0:r, _I:_I + 1] = xi * c + xj * s
        buf[0:r, _J:_J + 1] = xi * s - xj * c
        out_cp(i).start()
        for k in range(i + 1, _N):
            if _PREV[k] == i:
                out_cp(i).wait()
                in_cp(k).start()
    for i in range(_N):
        if all(_PREV[k] != i for k in range(i + 1, _N)):
            out_cp(i).wait()


def kernel(x, theta):
    theta_arr = jnp.reshape(theta, (1,)).astype(jnp.float32)
    return pl.pallas_call(
        _body,
        in_specs=[
            pl.BlockSpec(memory_space=pltpu.SMEM),
            pl.BlockSpec(memory_space=pl.ANY),
        ],
        out_specs=pl.BlockSpec(memory_space=pl.ANY),
        out_shape=jax.ShapeDtypeStruct((_ROWS, _COLS), jnp.float32),
        scratch_shapes=[
            pltpu.VMEM((_BUFSHAPES[0], _COLS), jnp.float32),
            pltpu.VMEM((_BUFSHAPES[1], _COLS), jnp.float32),
            pltpu.VMEM((_BUFSHAPES[2], _COLS), jnp.float32),
            pltpu.VMEM((_BUFSHAPES[3], _COLS), jnp.float32),
            pltpu.SemaphoreType.DMA((_N,)),
            pltpu.SemaphoreType.DMA((_N,)),
        ],
    )(theta_arr, x)
